# baseline (speedup 1.0000x reference)
import functools

import jax
import jax.numpy as jnp
from jax import lax
from jax.experimental import pallas as pl
from jax.experimental.pallas import tpu as pltpu
from jax.experimental.pallas import tpu_sc as plsc

ALPHA = 0.5
BETA = 0.5
S, C, H, W = 8, 4, 512, 512
NI = 4096
NC, NS, L = 2, 16, 16
NWORK = NC * NS
IV_PER_W = 2 * S * C * NI // NWORK
CHUNKS = IV_PER_W // L
BLK_CHUNKS = NI // L
DMA_CH = 128
N_DMA = IV_PER_W // DMA_CH

G0, G1 = 8, 4
WG0 = -ALPHA * BETA / (C * G0)
WB0 = ALPHA * (1.0 - BETA) / (C * (NI - G0))
WG1 = -(1.0 - ALPHA) * BETA / (C * G1)
WB1 = (1.0 - ALPHA) * (1.0 - BETA) / (C * (NI - G1))
CONST = (ALPHA + (1.0 - ALPHA)) * S * BETA


def _sc_body(pred_hbm, iv0_hbm, iv1_hbm, out_hbm,
             iv_v, idxb_v, idxd_v, valb_v, vald_v, acc_v, sem):
    cid = lax.axis_index("c")
    sid = lax.axis_index("s")
    wid = cid * NS + sid

    nints = IV_PER_W * 4

    @pl.when(cid == 0)
    def _():
        pltpu.sync_copy(iv0_hbm.at[pl.ds(sid * nints, nints)], iv_v)

    @pl.when(cid == 1)
    def _():
        pltpu.sync_copy(iv1_hbm.at[pl.ds(sid * nints, nints)], iv_v)

    iota = lax.iota(jnp.int32, L)

    def idx_body(j, carry):
        off = (j * L + iota) * 4
        bx = plsc.load_gather(iv_v, [off])
        by = plsc.load_gather(iv_v, [off + 1])
        dx = plsc.load_gather(iv_v, [off + 2])
        dy = plsc.load_gather(iv_v, [off + 3])
        blk = sid * 2 + j // BLK_CHUNKS
        base = blk * (H * W)
        idxb_v[pl.ds(j * L, L)] = base + bx * W + by
        idxd_v[pl.ds(j * L, L)] = base + dx * W + dy
        return carry

    lax.fori_loop(0, CHUNKS, idx_body, 0)

    def dma_body(k, carry):
        o = k * DMA_CH
        cb = pltpu.async_copy(pred_hbm.at[idxb_v.at[pl.ds(o, DMA_CH)]],
                              valb_v.at[pl.ds(o, DMA_CH)], sem)
        cd = pltpu.async_copy(pred_hbm.at[idxd_v.at[pl.ds(o, DMA_CH)]],
                              vald_v.at[pl.ds(o, DMA_CH)], sem)
        cb.wait()
        cd.wait()
        return carry

    lax.fori_loop(0, N_DMA, dma_body, 0)

    wg = jnp.where(cid == 0, WG0, WG1)
    wb = jnp.where(cid == 0, WB0, WB1)
    ng = jnp.where(cid == 0, G0, G1)

    def red_body(j, acc):
        b = valb_v[pl.ds(j * L, L)]
        d = vald_v[pl.ds(j * L, L)]
        pos = (j % BLK_CHUNKS) * L + iota
        wvec = jnp.where(pos < ng, wg, wb).astype(jnp.float32)
        df = b - d
        return acc + wvec * df * df

    acc = lax.fori_loop(0, CHUNKS, red_body, jnp.zeros((L,), jnp.float32))
    acc_v[...] = acc
    pltpu.sync_copy(acc_v, out_hbm.at[wid])


@functools.cache
def _interval_loss_sc():
    return pl.kernel(
        _sc_body,
        out_type=jax.ShapeDtypeStruct((NWORK, L), jnp.float32),
        mesh=plsc.VectorSubcoreMesh(core_axis_name="c", subcore_axis_name="s",
                                    num_cores=NC, num_subcores=NS),
        scratch_types=[
            pltpu.VMEM((IV_PER_W * 4,), jnp.int32),
            pltpu.VMEM((IV_PER_W,), jnp.int32),
            pltpu.VMEM((IV_PER_W,), jnp.int32),
            pltpu.VMEM((IV_PER_W,), jnp.float32),
            pltpu.VMEM((IV_PER_W,), jnp.float32),
            pltpu.VMEM((L,), jnp.float32),
            pltpu.SemaphoreType.DMA,
        ],
        compiler_params=pltpu.CompilerParams(needs_layout_passes=False),
    )


@jax.jit
def kernel(prediction, intervals_comp_0, intervals_comp_1):
    pred = prediction.reshape(-1)
    iv0 = intervals_comp_0.reshape(-1)
    iv1 = intervals_comp_1.reshape(-1)
    partials = _interval_loss_sc()(pred, iv0, iv1)
    return jnp.float32(CONST) + jnp.sum(partials)


# speedup vs baseline: 1.0463x; 1.0463x over previous
import functools

import jax
import jax.numpy as jnp
from jax import lax
from jax.experimental import pallas as pl
from jax.experimental.pallas import tpu as pltpu
from jax.experimental.pallas import tpu_sc as plsc

ALPHA = 0.5
BETA = 0.5
S, C, H, W = 8, 4, 512, 512
NI = 4096
NC, NS, L = 2, 16, 16
NWORK = NC * NS
IV_PER_W = 2 * S * C * NI // NWORK
CHUNKS = IV_PER_W // L
BLK_CHUNKS = NI // L
DMA_CH = 128
N_DMA = IV_PER_W // DMA_CH

G0, G1 = 8, 4
WG0 = -ALPHA * BETA / (C * G0)
WB0 = ALPHA * (1.0 - BETA) / (C * (NI - G0))
WG1 = -(1.0 - ALPHA) * BETA / (C * G1)
WB1 = (1.0 - ALPHA) * (1.0 - BETA) / (C * (NI - G1))
CONST = (ALPHA + (1.0 - ALPHA)) * S * BETA


def _sc_body(pred_hbm, iv0_hbm, iv1_hbm, out_hbm,
             iv_v, idxb_v, idxd_v, valb_v, vald_v, acc_v, sem):
    cid = lax.axis_index("c")
    sid = lax.axis_index("s")
    wid = cid * NS + sid

    nints = IV_PER_W * 4

    @pl.when(cid == 0)
    def _():
        pltpu.sync_copy(iv0_hbm.at[pl.ds(sid * nints, nints)], iv_v)

    @pl.when(cid == 1)
    def _():
        pltpu.sync_copy(iv1_hbm.at[pl.ds(sid * nints, nints)], iv_v)

    iota = lax.iota(jnp.int32, L)
    sub_per_row = DMA_CH // L

    def row_body(k, carry):
        for sub in range(sub_per_row):
            j = k * sub_per_row + sub
            off = (j * L + iota) * 4
            bx = plsc.load_gather(iv_v, [off])
            by = plsc.load_gather(iv_v, [off + 1])
            dx = plsc.load_gather(iv_v, [off + 2])
            dy = plsc.load_gather(iv_v, [off + 3])
            blk = sid * 2 + j // BLK_CHUNKS
            base = blk * (H * W)
            idxb_v[k, pl.ds(sub * L, L)] = base + bx * W + by
            idxd_v[k, pl.ds(sub * L, L)] = base + dx * W + dy
        pltpu.async_copy(pred_hbm.at[idxb_v.at[k]],
                         valb_v.at[pl.ds(k * DMA_CH, DMA_CH)], sem)
        pltpu.async_copy(pred_hbm.at[idxd_v.at[k]],
                         vald_v.at[pl.ds(k * DMA_CH, DMA_CH)], sem)
        return carry

    lax.fori_loop(0, N_DMA, row_body, 0)
    pltpu.make_async_copy(pred_hbm.at[pl.ds(0, IV_PER_W)], valb_v, sem).wait()
    pltpu.make_async_copy(pred_hbm.at[pl.ds(0, IV_PER_W)], vald_v, sem).wait()

    wg = jnp.where(cid == 0, WG0, WG1)
    wb = jnp.where(cid == 0, WB0, WB1)
    ng = jnp.where(cid == 0, G0, G1)

    def red_body(j, acc):
        b = valb_v[pl.ds(j * L, L)]
        d = vald_v[pl.ds(j * L, L)]
        pos = (j % BLK_CHUNKS) * L + iota
        wvec = jnp.where(pos < ng, wg, wb).astype(jnp.float32)
        df = b - d
        return acc + wvec * df * df

    acc = lax.fori_loop(0, CHUNKS, red_body, jnp.zeros((L,), jnp.float32))
    acc_v[...] = acc
    pltpu.sync_copy(acc_v, out_hbm.at[wid])


@functools.cache
def _interval_loss_sc():
    return pl.kernel(
        _sc_body,
        out_type=jax.ShapeDtypeStruct((NWORK, L), jnp.float32),
        mesh=plsc.VectorSubcoreMesh(core_axis_name="c", subcore_axis_name="s",
                                    num_cores=NC, num_subcores=NS),
        scratch_types=[
            pltpu.VMEM((IV_PER_W * 4,), jnp.int32),
            pltpu.VMEM((N_DMA, DMA_CH), jnp.int32),
            pltpu.VMEM((N_DMA, DMA_CH), jnp.int32),
            pltpu.VMEM((IV_PER_W,), jnp.float32),
            pltpu.VMEM((IV_PER_W,), jnp.float32),
            pltpu.VMEM((L,), jnp.float32),
            pltpu.SemaphoreType.DMA,
        ],
        compiler_params=pltpu.CompilerParams(needs_layout_passes=False),
    )


@jax.jit
def kernel(prediction, intervals_comp_0, intervals_comp_1):
    pred = prediction.reshape(-1)
    iv0 = intervals_comp_0.reshape(-1)
    iv1 = intervals_comp_1.reshape(-1)
    partials = _interval_loss_sc()(pred, iv0, iv1)
    return jnp.float32(CONST) + jnp.sum(partials)


# speedup vs baseline: 12.4919x; 11.9392x over previous
import functools

import jax
import jax.numpy as jnp
from jax import lax
from jax.experimental import pallas as pl
from jax.experimental.pallas import tpu as pltpu
from jax.experimental.pallas import tpu_sc as plsc

ALPHA = 0.5
BETA = 0.5
S, C, H, W = 8, 4, 512, 512
NI = 4096
NC, NS, L = 2, 16, 16
NWORK = NC * NS
IV_PER_W = 2 * S * C * NI // NWORK
CHUNKS = IV_PER_W // L
BLK_CHUNKS = NI // L
DMA_CH = 128
N_DMA = IV_PER_W // DMA_CH

G0, G1 = 8, 4
WG0 = -ALPHA * BETA / (C * G0)
WB0 = ALPHA * (1.0 - BETA) / (C * (NI - G0))
WG1 = -(1.0 - ALPHA) * BETA / (C * G1)
WB1 = (1.0 - ALPHA) * (1.0 - BETA) / (C * (NI - G1))
CONST = (ALPHA + (1.0 - ALPHA)) * S * BETA


def _sc_body(pred_hbm, bx0_hbm, by0_hbm, dx0_hbm, dy0_hbm,
             bx1_hbm, by1_hbm, dx1_hbm, dy1_hbm, out_hbm,
             bx_v, by_v, dx_v, dy_v, idxb_v, idxd_v, valb_v, vald_v,
             acc_v, sem):
    cid = lax.axis_index("c")
    sid = lax.axis_index("s")
    wid = cid * NS + sid

    span = pl.ds(sid * IV_PER_W, IV_PER_W)

    @pl.when(cid == 0)
    def _():
        pltpu.sync_copy(bx0_hbm.at[span], bx_v)
        pltpu.sync_copy(by0_hbm.at[span], by_v)
        pltpu.sync_copy(dx0_hbm.at[span], dx_v)
        pltpu.sync_copy(dy0_hbm.at[span], dy_v)

    @pl.when(cid == 1)
    def _():
        pltpu.sync_copy(bx1_hbm.at[span], bx_v)
        pltpu.sync_copy(by1_hbm.at[span], by_v)
        pltpu.sync_copy(dx1_hbm.at[span], dx_v)
        pltpu.sync_copy(dy1_hbm.at[span], dy_v)

    iota = lax.iota(jnp.int32, L)
    blk0 = sid * 2
    sub_per_row = DMA_CH // L

    def row_body(k, carry):
        for sub in range(sub_per_row):
            j = k * sub_per_row + sub
            o = pl.ds(j * L, L)
            base = (blk0 + j // BLK_CHUNKS) * (H * W)
            idxb_v[k, pl.ds(sub * L, L)] = base + bx_v[o] * W + by_v[o]
            idxd_v[k, pl.ds(sub * L, L)] = base + dx_v[o] * W + dy_v[o]
        pltpu.async_copy(pred_hbm.at[idxb_v.at[k]],
                         valb_v.at[pl.ds(k * DMA_CH, DMA_CH)], sem)
        pltpu.async_copy(pred_hbm.at[idxd_v.at[k]],
                         vald_v.at[pl.ds(k * DMA_CH, DMA_CH)], sem)
        return carry

    lax.fori_loop(0, N_DMA, row_body, 0)
    pltpu.make_async_copy(pred_hbm.at[pl.ds(0, IV_PER_W)], valb_v, sem).wait()
    pltpu.make_async_copy(pred_hbm.at[pl.ds(0, IV_PER_W)], vald_v, sem).wait()

    wg = jnp.where(cid == 0, WG0, WG1)
    wb = jnp.where(cid == 0, WB0, WB1)
    ng = jnp.where(cid == 0, G0, G1)

    def red_body(j, acc):
        b = valb_v[pl.ds(j * L, L)]
        d = vald_v[pl.ds(j * L, L)]
        pos = (j % BLK_CHUNKS) * L + iota
        wvec = jnp.where(pos < ng, wg, wb).astype(jnp.float32)
        df = b - d
        return acc + wvec * df * df

    acc = lax.fori_loop(0, CHUNKS, red_body, jnp.zeros((L,), jnp.float32))
    acc_v[...] = acc
    pltpu.sync_copy(acc_v, out_hbm.at[wid])


@functools.cache
def _interval_loss_sc():
    return pl.kernel(
        _sc_body,
        out_type=jax.ShapeDtypeStruct((NWORK, L), jnp.float32),
        mesh=plsc.VectorSubcoreMesh(core_axis_name="c", subcore_axis_name="s",
                                    num_cores=NC, num_subcores=NS),
        scratch_types=[
            pltpu.VMEM((IV_PER_W,), jnp.int32),
            pltpu.VMEM((IV_PER_W,), jnp.int32),
            pltpu.VMEM((IV_PER_W,), jnp.int32),
            pltpu.VMEM((IV_PER_W,), jnp.int32),
            pltpu.VMEM((N_DMA, DMA_CH), jnp.int32),
            pltpu.VMEM((N_DMA, DMA_CH), jnp.int32),
            pltpu.VMEM((IV_PER_W,), jnp.float32),
            pltpu.VMEM((IV_PER_W,), jnp.float32),
            pltpu.VMEM((L,), jnp.float32),
            pltpu.SemaphoreType.DMA,
        ],
        compiler_params=pltpu.CompilerParams(needs_layout_passes=False),
    )


@jax.jit
def kernel(prediction, intervals_comp_0, intervals_comp_1):
    pred = prediction.reshape(-1)
    planes = []
    for iv in (intervals_comp_0, intervals_comp_1):
        for a, b in ((0, 0), (0, 1), (1, 0), (1, 1)):
            planes.append(iv[:, :, :, a, b].reshape(-1))
    partials = _interval_loss_sc()(pred, *planes)
    return jnp.float32(CONST) + jnp.sum(partials)


# speedup vs baseline: 12.6997x; 1.0166x over previous
import functools

import jax
import jax.numpy as jnp
from jax import lax
from jax.experimental import pallas as pl
from jax.experimental.pallas import tpu as pltpu
from jax.experimental.pallas import tpu_sc as plsc

ALPHA = 0.5
BETA = 0.5
S, C, H, W = 8, 4, 512, 512
NI = 4096
NC, NS, L = 2, 16, 16
NWORK = NC * NS
IV_PER_W = 2 * S * C * NI // NWORK
CHUNKS = IV_PER_W // L
BLK_CHUNKS = NI // L
DMA_CH = 128
N_DMA = IV_PER_W // DMA_CH

G0, G1 = 8, 4
WG0 = -ALPHA * BETA / (C * G0)
WB0 = ALPHA * (1.0 - BETA) / (C * (NI - G0))
WG1 = -(1.0 - ALPHA) * BETA / (C * G1)
WB1 = (1.0 - ALPHA) * (1.0 - BETA) / (C * (NI - G1))
CONST = (ALPHA + (1.0 - ALPHA)) * S * BETA


def _sc_body(pred_hbm, bx0_hbm, by0_hbm, dx0_hbm, dy0_hbm,
             bx1_hbm, by1_hbm, dx1_hbm, dy1_hbm, out_hbm,
             bx_v, by_v, dx_v, dy_v, idxb_v, idxd_v, valb_v, vald_v,
             acc_v, sem):
    cid = lax.axis_index("c")
    sid = lax.axis_index("s")
    wid = cid * NS + sid

    span = pl.ds(sid * IV_PER_W, IV_PER_W)

    @pl.when(cid == 0)
    def _():
        pltpu.sync_copy(bx0_hbm.at[span], bx_v)
        pltpu.sync_copy(by0_hbm.at[span], by_v)
        pltpu.sync_copy(dx0_hbm.at[span], dx_v)
        pltpu.sync_copy(dy0_hbm.at[span], dy_v)

    @pl.when(cid == 1)
    def _():
        pltpu.sync_copy(bx1_hbm.at[span], bx_v)
        pltpu.sync_copy(by1_hbm.at[span], by_v)
        pltpu.sync_copy(dx1_hbm.at[span], dx_v)
        pltpu.sync_copy(dy1_hbm.at[span], dy_v)

    iota = lax.iota(jnp.int32, L)
    blk0 = sid * 2
    sub_per_row = DMA_CH // L

    def row_body(k, carry):
        for sub in range(sub_per_row):
            j = k * sub_per_row + sub
            o = pl.ds(j * L, L)
            base = (blk0 + j // BLK_CHUNKS) * (H * W)
            idxb_v[k, pl.ds(sub * L, L)] = base + bx_v[o] * W + by_v[o]
            idxd_v[k, pl.ds(sub * L, L)] = base + dx_v[o] * W + dy_v[o]
        pltpu.async_copy(pred_hbm.at[idxb_v.at[k]],
                         valb_v.at[pl.ds(k * DMA_CH, DMA_CH)], sem)
        pltpu.async_copy(pred_hbm.at[idxd_v.at[k]],
                         vald_v.at[pl.ds(k * DMA_CH, DMA_CH)], sem)
        return carry

    lax.fori_loop(0, N_DMA, row_body, 0)
    pltpu.make_async_copy(pred_hbm.at[pl.ds(0, IV_PER_W)], valb_v, sem).wait()
    pltpu.make_async_copy(pred_hbm.at[pl.ds(0, IV_PER_W)], vald_v, sem).wait()

    wg = jnp.where(cid == 0, WG0, WG1)
    wb = jnp.where(cid == 0, WB0, WB1)
    ng = jnp.where(cid == 0, G0, G1)

    def red_body(k, acc):
        for sub in range(sub_per_row):
            o = pl.ds((k * sub_per_row + sub) * L, L)
            df = valb_v[o] - vald_v[o]
            acc = acc + df * df
        return acc

    acc = lax.fori_loop(0, N_DMA, red_body, jnp.zeros((L,), jnp.float32))
    acc = acc * wb.astype(jnp.float32)
    dw = (wg - wb).astype(jnp.float32)
    for t in range(2):
        o = pl.ds(t * NI, L)
        df = valb_v[o] - vald_v[o]
        acc = acc + jnp.where(iota < ng, dw * df * df, 0.0)
    acc_v[...] = acc
    pltpu.sync_copy(acc_v, out_hbm.at[wid])


@functools.cache
def _interval_loss_sc():
    return pl.kernel(
        _sc_body,
        out_type=jax.ShapeDtypeStruct((NWORK, L), jnp.float32),
        mesh=plsc.VectorSubcoreMesh(core_axis_name="c", subcore_axis_name="s",
                                    num_cores=NC, num_subcores=NS),
        scratch_types=[
            pltpu.VMEM((IV_PER_W,), jnp.int32),
            pltpu.VMEM((IV_PER_W,), jnp.int32),
            pltpu.VMEM((IV_PER_W,), jnp.int32),
            pltpu.VMEM((IV_PER_W,), jnp.int32),
            pltpu.VMEM((N_DMA, DMA_CH), jnp.int32),
            pltpu.VMEM((N_DMA, DMA_CH), jnp.int32),
            pltpu.VMEM((IV_PER_W,), jnp.float32),
            pltpu.VMEM((IV_PER_W,), jnp.float32),
            pltpu.VMEM((L,), jnp.float32),
            pltpu.SemaphoreType.DMA,
        ],
        compiler_params=pltpu.CompilerParams(needs_layout_passes=False),
    )


@jax.jit
def kernel(prediction, intervals_comp_0, intervals_comp_1):
    pred = prediction.reshape(-1)
    planes = []
    for iv in (intervals_comp_0, intervals_comp_1):
        for a, b in ((0, 0), (0, 1), (1, 0), (1, 1)):
            planes.append(iv[:, :, :, a, b].reshape(-1))
    partials = _interval_loss_sc()(pred, *planes)
    return jnp.float32(CONST) + jnp.sum(partials)


# speedup vs baseline: 13.0135x; 1.0247x over previous
import functools

import jax
import jax.numpy as jnp
from jax import lax
from jax.experimental import pallas as pl
from jax.experimental.pallas import tpu as pltpu
from jax.experimental.pallas import tpu_sc as plsc

ALPHA = 0.5
BETA = 0.5
S, C, H, W = 8, 4, 512, 512
NI = 4096
NC, NS, L = 2, 16, 16
NWORK = NC * NS
IV_PER_W = 2 * S * C * NI // NWORK
CHUNKS = IV_PER_W // L
BLK_CHUNKS = NI // L
DMA_CH = 128
N_DMA = IV_PER_W // DMA_CH

G0, G1 = 8, 4
WG0 = -ALPHA * BETA / (C * G0)
WB0 = ALPHA * (1.0 - BETA) / (C * (NI - G0))
WG1 = -(1.0 - ALPHA) * BETA / (C * G1)
WB1 = (1.0 - ALPHA) * (1.0 - BETA) / (C * (NI - G1))
CONST = (ALPHA + (1.0 - ALPHA)) * S * BETA


def _sc_body(pred_hbm, bx0_hbm, by0_hbm, dx0_hbm, dy0_hbm,
             bx1_hbm, by1_hbm, dx1_hbm, dy1_hbm, out_hbm,
             bx_v, by_v, dx_v, dy_v, idxb_v, idxd_v, valb_v, vald_v,
             acc_v, sem):
    cid = lax.axis_index("c")
    sid = lax.axis_index("s")
    wid = cid * NS + sid

    span = pl.ds(sid * IV_PER_W, IV_PER_W)

    @pl.when(cid == 0)
    def _():
        pltpu.async_copy(bx0_hbm.at[span], bx_v, sem)
        pltpu.async_copy(by0_hbm.at[span], by_v, sem)
        pltpu.async_copy(dx0_hbm.at[span], dx_v, sem)
        pltpu.async_copy(dy0_hbm.at[span], dy_v, sem)

    @pl.when(cid == 1)
    def _():
        pltpu.async_copy(bx1_hbm.at[span], bx_v, sem)
        pltpu.async_copy(by1_hbm.at[span], by_v, sem)
        pltpu.async_copy(dx1_hbm.at[span], dx_v, sem)
        pltpu.async_copy(dy1_hbm.at[span], dy_v, sem)

    pltpu.make_async_copy(bx0_hbm.at[span], bx_v, sem).wait()
    pltpu.make_async_copy(bx0_hbm.at[span], by_v, sem).wait()
    pltpu.make_async_copy(bx0_hbm.at[span], dx_v, sem).wait()
    pltpu.make_async_copy(bx0_hbm.at[span], dy_v, sem).wait()

    iota = lax.iota(jnp.int32, L)
    blk0 = sid * 2
    sub_per_row = DMA_CH // L

    def row_body(k, carry):
        for sub in range(sub_per_row):
            j = k * sub_per_row + sub
            o = pl.ds(j * L, L)
            base = (blk0 + j // BLK_CHUNKS) * (H * W)
            idxb_v[k, pl.ds(sub * L, L)] = base + bx_v[o] * W + by_v[o]
            idxd_v[k, pl.ds(sub * L, L)] = base + dx_v[o] * W + dy_v[o]
        pltpu.async_copy(pred_hbm.at[idxb_v.at[k]],
                         valb_v.at[pl.ds(k * DMA_CH, DMA_CH)], sem)
        pltpu.async_copy(pred_hbm.at[idxd_v.at[k]],
                         vald_v.at[pl.ds(k * DMA_CH, DMA_CH)], sem)
        return carry

    lax.fori_loop(0, N_DMA, row_body, 0)
    pltpu.make_async_copy(pred_hbm.at[pl.ds(0, IV_PER_W)], valb_v, sem).wait()
    pltpu.make_async_copy(pred_hbm.at[pl.ds(0, IV_PER_W)], vald_v, sem).wait()

    wg = jnp.where(cid == 0, WG0, WG1)
    wb = jnp.where(cid == 0, WB0, WB1)
    ng = jnp.where(cid == 0, G0, G1)

    def red_body(k, acc):
        for sub in range(sub_per_row):
            o = pl.ds((k * sub_per_row + sub) * L, L)
            df = valb_v[o] - vald_v[o]
            acc = acc + df * df
        return acc

    acc = lax.fori_loop(0, N_DMA, red_body, jnp.zeros((L,), jnp.float32))
    acc = acc * wb.astype(jnp.float32)
    dw = (wg - wb).astype(jnp.float32)
    for t in range(2):
        o = pl.ds(t * NI, L)
        df = valb_v[o] - vald_v[o]
        acc = acc + jnp.where(iota < ng, dw * df * df, 0.0)
    acc_v[...] = acc
    pltpu.sync_copy(acc_v, out_hbm.at[wid])


@functools.cache
def _interval_loss_sc():
    return pl.kernel(
        _sc_body,
        out_type=jax.ShapeDtypeStruct((NWORK, L), jnp.float32),
        mesh=plsc.VectorSubcoreMesh(core_axis_name="c", subcore_axis_name="s",
                                    num_cores=NC, num_subcores=NS),
        scratch_types=[
            pltpu.VMEM((IV_PER_W,), jnp.int32),
            pltpu.VMEM((IV_PER_W,), jnp.int32),
            pltpu.VMEM((IV_PER_W,), jnp.int32),
            pltpu.VMEM((IV_PER_W,), jnp.int32),
            pltpu.VMEM((N_DMA, DMA_CH), jnp.int32),
            pltpu.VMEM((N_DMA, DMA_CH), jnp.int32),
            pltpu.VMEM((IV_PER_W,), jnp.float32),
            pltpu.VMEM((IV_PER_W,), jnp.float32),
            pltpu.VMEM((L,), jnp.float32),
            pltpu.SemaphoreType.DMA,
        ],
        compiler_params=pltpu.CompilerParams(needs_layout_passes=False),
    )


@jax.jit
def kernel(prediction, intervals_comp_0, intervals_comp_1):
    pred = prediction.reshape(-1)
    planes = []
    for iv in (intervals_comp_0, intervals_comp_1):
        for a, b in ((0, 0), (0, 1), (1, 0), (1, 1)):
            planes.append(iv[:, :, :, a, b].reshape(-1))
    partials = _interval_loss_sc()(pred, *planes)
    return jnp.float32(CONST) + jnp.sum(partials)
